# baseline (speedup 1.0000x reference)
import functools

import jax
import jax.numpy as jnp
import numpy as np
from jax.experimental import pallas as pl
from jax.experimental.pallas import tpu as pltpu

MUL = 16
RADIAL = 8
CUTOFF = 5.0
EPS = 0.25


def _edge_body(v_ref, snd_ref, w1_ref, w2_ref, w3_ref, out_ref):
    v = v_ref[...]
    x = v[:, 0:1]
    y = v[:, 1:2]
    z = v[:, 2:3]
    r2 = x * x + y * y + z * z
    r = jnp.sqrt(r2)
    u = r * (1.0 / CUTOFF)
    u2 = u * u
    u4 = u2 * u2
    up = u4 * u
    env = 1.0 - 21.0 * up + 35.0 * up * u - 15.0 * up * u2
    env = jnp.where(r < CUTOFF, env, 0.0)
    ns = jax.lax.broadcasted_iota(jnp.int32, (1, RADIAL), 1).astype(jnp.float32) + 1.0
    xa = ns * u
    px = jnp.pi * xa
    sinc = jnp.where(xa > 0.0, jnp.sin(px) / jnp.where(xa > 0.0, px, 1.0), 1.0)
    bessel = (np.sqrt(2.0 / CUTOFF) * jnp.pi / CUTOFF) * ns * sinc
    rbf = bessel * env
    h = rbf @ w1_ref[...]
    h = h * jax.nn.sigmoid(h)
    h = h @ w2_ref[...]
    h = h * jax.nn.sigmoid(h)
    tpw = h @ w3_ref[...]
    rinv = jnp.sqrt(3.0) / (r + 1e-12)
    y1 = v * rinv
    snd = snd_ref[...].astype(jnp.float32)
    out_ref[:, 0:64] = tpw
    out_ref[:, 64:67] = y1
    out_ref[:, 67:68] = snd


def _edge_stage(vectors, senders, mlp_w1, mlp_w2, w3s, block_e):
    e = vectors.shape[0]
    grid = e // block_e
    return pl.pallas_call(
        _edge_body,
        grid=(grid,),
        in_specs=[
            pl.BlockSpec((block_e, 3), lambda i: (i, 0)),
            pl.BlockSpec((block_e, 1), lambda i: (i, 0)),
            pl.BlockSpec((RADIAL, 64), lambda i: (0, 0)),
            pl.BlockSpec((64, 64), lambda i: (0, 0)),
            pl.BlockSpec((64, 64), lambda i: (0, 0)),
        ],
        out_specs=pl.BlockSpec((block_e, 68), lambda i: (i, 0)),
        out_shape=jax.ShapeDtypeStruct((e, 68), jnp.float32),
    )(vectors, senders.reshape(e, 1), mlp_w1, mlp_w2, w3s)


def _node_body(cs_ref, cv_ref, ns_ref, nv_ref, sp_ref,
               w0_ref, w1_ref, sk0_ref, sk1_ref, outs_ref, outv_ref):
    cs = cs_ref[...]
    sg = cs @ w0_ref[...]
    scal = ns_ref[...]
    spf = sp_ref[...].astype(jnp.float32)
    for k in range(4):
        mk = jnp.where(spf == float(k), 1.0, 0.0)
        sg = sg + mk * (scal @ sk0_ref[k])
    gates = jax.nn.sigmoid(sg[:, MUL:2 * MUL])
    s_out = sg[:, 0:MUL]
    outs_ref[...] = s_out * jax.nn.sigmoid(s_out)
    cv = cv_ref[...]
    nv = nv_ref[...]
    for i in range(3):
        vi = cv[:, 32 * i:32 * i + 32] @ w1_ref[...]
        for k in range(4):
            mk = jnp.where(spf == float(k), 1.0, 0.0)
            vi = vi + mk * (nv[:, 16 * i:16 * i + 16] @ sk1_ref[k])
        outv_ref[:, 16 * i:16 * i + 16] = vi * gates


def _node_stage(conv_s, conv_v96, node_scalars, nv48, species,
                w0s, w1s, sk0s, sk1s, block_n):
    n = conv_s.shape[0]
    grid = n // block_n
    return pl.pallas_call(
        _node_body,
        grid=(grid,),
        in_specs=[
            pl.BlockSpec((block_n, 32), lambda i: (i, 0)),
            pl.BlockSpec((block_n, 96), lambda i: (i, 0)),
            pl.BlockSpec((block_n, 16), lambda i: (i, 0)),
            pl.BlockSpec((block_n, 48), lambda i: (i, 0)),
            pl.BlockSpec((block_n, 1), lambda i: (i, 0)),
            pl.BlockSpec((32, 32), lambda i: (0, 0)),
            pl.BlockSpec((32, 16), lambda i: (0, 0)),
            pl.BlockSpec((4, 16, 32), lambda i: (0, 0, 0)),
            pl.BlockSpec((4, 16, 16), lambda i: (0, 0, 0)),
        ],
        out_specs=[
            pl.BlockSpec((block_n, 16), lambda i: (i, 0)),
            pl.BlockSpec((block_n, 48), lambda i: (i, 0)),
        ],
        out_shape=[
            jax.ShapeDtypeStruct((n, 16), jnp.float32),
            jax.ShapeDtypeStruct((n, 48), jnp.float32),
        ],
    )(conv_s, conv_v96, node_scalars, nv48, species.reshape(n, 1),
      w0s, w1s, sk0s, sk1s)


def _pick_block(total, want):
    b = min(want, total)
    while total % b:
        b -= 1
    return b


def kernel(node_scalars, node_vectors, vectors, species, senders, receivers,
           mlp_w1, mlp_w2, mlp_w3, lin_w0e, lin_w1o, skip_w0e, skip_w1o):
    e = vectors.shape[0]
    n = node_scalars.shape[0]

    isq3 = 1.0 / np.sqrt(3.0)
    path_scale = np.repeat(np.array([1.0, isq3, 1.0, isq3], np.float32), MUL)
    w3s = mlp_w3 * (EPS * path_scale)[None, :]

    block_e = _pick_block(e, 2000)
    edgedata = _edge_stage(vectors, senders, mlp_w1, mlp_w2, w3s, block_e)

    tpw = edgedata[:, 0:64]
    y1 = edgedata[:, 64:67]
    s_send = node_scalars[senders]
    v_send = node_vectors[senders]
    m0_a = tpw[:, 0:16] * s_send
    m0_b = tpw[:, 48:64] * jnp.einsum('emi,ei->em', v_send, y1)
    m1_a = tpw[:, 16:32][:, :, None] * s_send[:, :, None] * y1[:, None, :]
    m1_b = tpw[:, 32:48][:, :, None] * v_send
    msg_s = jnp.concatenate([m0_a, m0_b], axis=1)
    msg_v = jnp.concatenate([m1_a, m1_b], axis=1)
    conv_s = jnp.zeros((n, 32), jnp.float32).at[receivers].add(msg_s)
    conv_v = jnp.zeros((n, 32, 3), jnp.float32).at[receivers].add(msg_v)

    conv_v96 = conv_v.transpose(0, 2, 1).reshape(n, 96)
    nv48 = node_vectors.transpose(0, 2, 1).reshape(n, 48)
    rs2 = np.float32(1.0 / np.sqrt(2.0 * MUL))
    rs1 = np.float32(1.0 / np.sqrt(1.0 * MUL))
    block_n = _pick_block(n, 2000)
    out_s, out_v48 = _node_stage(
        conv_s, conv_v96, node_scalars, nv48, species,
        lin_w0e * rs2, lin_w1o * rs2, skip_w0e * rs1, skip_w1o * rs1,
        block_n)
    out_v = out_v48.reshape(n, 3, MUL).transpose(0, 2, 1)
    return out_s, out_v


# speedup vs baseline: 22.7742x; 22.7742x over previous
import functools

import jax
import jax.numpy as jnp
import numpy as np
from jax import lax
from jax.experimental import pallas as pl
from jax.experimental.pallas import tpu as pltpu
from jax.experimental.pallas import tpu_sc as plsc

MUL = 16
RADIAL = 8
CUTOFF = 5.0
EPS = 0.25

NB = 8
K = 128
ROW = 72
NDROW = 64


def _edge_body(v_ref, snd_ref, w1_ref, w2_ref, w3_ref, out_ref):
    v = v_ref[...]
    x = v[:, 0:1]
    y = v[:, 1:2]
    z = v[:, 2:3]
    r2 = x * x + y * y + z * z
    r = jnp.sqrt(r2)
    u = r * (1.0 / CUTOFF)
    u2 = u * u
    u4 = u2 * u2
    up = u4 * u
    env = 1.0 - 21.0 * up + 35.0 * up * u - 15.0 * up * u2
    env = jnp.where(r < CUTOFF, env, 0.0)
    ns = lax.broadcasted_iota(jnp.int32, (1, RADIAL), 1).astype(jnp.float32) + 1.0
    xa = ns * u
    px = jnp.pi * xa
    sinc = jnp.where(xa > 0.0, jnp.sin(px) / jnp.where(xa > 0.0, px, 1.0), 1.0)
    bessel = (np.sqrt(2.0 / CUTOFF) * jnp.pi / CUTOFF) * ns * sinc
    rbf = bessel * env
    h = rbf @ w1_ref[...]
    h = h * jax.nn.sigmoid(h)
    h = h @ w2_ref[...]
    h = h * jax.nn.sigmoid(h)
    tpw = h @ w3_ref[...]
    rinv = jnp.sqrt(3.0) / (r + 1e-12)
    y1 = v * rinv
    snd = snd_ref[...].astype(jnp.float32)
    out_ref[:, 0:64] = tpw
    out_ref[:, 64:67] = y1
    out_ref[:, 67:68] = snd
    out_ref[:, 68:72] = jnp.zeros_like(tpw[:, 0:4])


def _edge_stage(vectors, senders, mlp_w1, mlp_w2, w3s, block_e):
    e = vectors.shape[0]
    grid = e // block_e
    return pl.pallas_call(
        _edge_body,
        grid=(grid,),
        in_specs=[
            pl.BlockSpec((block_e, 3), lambda i: (i, 0)),
            pl.BlockSpec((block_e, 1), lambda i: (i, 0)),
            pl.BlockSpec((RADIAL, 64), lambda i: (0, 0)),
            pl.BlockSpec((64, 64), lambda i: (0, 0)),
            pl.BlockSpec((64, 64), lambda i: (0, 0)),
        ],
        out_specs=pl.BlockSpec((block_e, ROW), lambda i: (i, 0)),
        out_shape=jax.ShapeDtypeStruct((e, ROW), jnp.float32),
    )(vectors, senders.reshape(e, 1), mlp_w1, mlp_w2, w3s)


def _sc_conv(edgedata, receivers, nodedata):
    e = edgedata.shape[0]
    n = nodedata.shape[0]
    info = plsc.get_sparse_core_info()
    nc, ns = info.num_cores, info.num_subcores
    nw = nc * ns
    epw = e // nw
    assert epw * nw == e
    quant = NB * ns * 8
    npad = ((n + quant - 1) // quant) * quant
    rng = npad // NB
    rows_pt = rng // ns
    zrows = max(z for z in range(8, 129, 8) if rows_pt % z == 0)
    assert rows_pt % zrows == 0 and zrows % 8 == 0
    cchunk = 5000
    assert epw % cchunk == 0 and cchunk % 8 == 0
    nchunks = epw // cchunk
    nfull = cchunk // 16
    rem = cchunk - nfull * 16
    cap = ((epw + NB * 16 + K + 15) // 16) * 16

    mesh = plsc.VectorSubcoreMesh(core_axis_name="c", subcore_axis_name="s")

    @functools.partial(
        pl.kernel,
        out_type=jax.ShapeDtypeStruct((nc, npad, 128), jnp.float32),
        mesh=mesh,
        compiler_params=pltpu.CompilerParams(needs_layout_passes=False, use_tc_tiling_on_sc=False),
        scratch_types=[
            pltpu.VMEM((cchunk + 16,), jnp.int32),
            pltpu.VMEM((cap,), jnp.int32),
            pltpu.VMEM((K,), jnp.int32),
            pltpu.VMEM((K,), jnp.int32),
            pltpu.VMEM((K,), jnp.int32),
            pltpu.VMEM((K, ROW), jnp.float32),
            pltpu.VMEM((K, NDROW), jnp.float32),
            pltpu.VMEM((K, 128), jnp.float32),
            pltpu.VMEM((zrows, 128), jnp.float32),
            pltpu.VMEM_SHARED((rng, 128), jnp.float32),
            pltpu.SemaphoreType.DMA,
        ],
    )
    def body(edata, recv, ndata, out, rbuf, plist, eidc, lrecc, sidx,
             ebuf, nbuf, msgbuf, zbuf, acc, sem):
        cid = lax.axis_index("c")
        sid = lax.axis_index("s")
        wid = cid * ns + sid
        ebase = wid * epw
        iot = lax.broadcasted_iota(jnp.int32, (16,), 0)
        zero16 = jnp.zeros((16,), jnp.float32)
        zi16 = jnp.zeros((16,), jnp.int32)

        def _zrow(r, c):
            for g in range(8):
                zbuf[r, 16 * g:16 * g + 16] = zero16
            return c
        lax.fori_loop(0, zrows, _zrow, 0)

        def _zlist(r, c):
            plist[pl.ds(r * 16, 16)] = zi16
            return c
        lax.fori_loop(0, cap // 16, _zlist, 0)

        cnts = tuple(jnp.int32(0) for _ in range(NB))
        for c in range(nchunks):
            pltpu.sync_copy(recv.at[pl.ds(ebase + c * cchunk, cchunk)],
                            rbuf.at[pl.ds(0, cchunk)])

            def _cnt(j, carry):
                rv = rbuf[pl.ds(j * 16, 16)]
                res = []
                for b in range(NB):
                    m = (rv >= b * rng) & (rv < (b + 1) * rng)
                    res.append(carry[b] + jnp.sum(m.astype(jnp.int32)))
                return tuple(res)
            cnts = lax.fori_loop(0, nfull, _cnt, cnts)
            rv = rbuf[pl.ds(nfull * 16, 16)]
            mv = iot < rem
            cnts = tuple(
                cnts[b] + jnp.sum(
                    ((rv >= b * rng) & (rv < (b + 1) * rng) & mv)
                    .astype(jnp.int32))
                for b in range(NB))

        cnt_s = list(cnts)
        start_s = []
        cur = jnp.int32(0)
        for b in range(NB):
            start_s.append(cur)
            cur = lax.bitwise_and(cur + cnt_s[b] + 15, jnp.int32(-16))

        ptrs = tuple(jnp.full((16,), start_s[b], jnp.int32) for b in range(NB))
        for c in range(nchunks):
            pltpu.sync_copy(recv.at[pl.ds(ebase + c * cchunk, cchunk)],
                            rbuf.at[pl.ds(0, cchunk)])

            def _scatter_vreg(rv, ev, extra_mask, ptr):
                res = []
                for b in range(NB):
                    m = (rv >= b * rng) & (rv < (b + 1) * rng)
                    if extra_mask is not None:
                        m = m & extra_mask
                    ranks = plsc.cumsum(m.astype(jnp.int32))
                    pos = ptr[b] + ranks - 1
                    packed = lax.shift_left(ev, 13) | (rv - b * rng)
                    plsc.store_scatter(plist, [pos], packed, mask=m)
                    res.append(ptr[b] + jnp.sum(m.astype(jnp.int32)))
                return tuple(res)

            def _part(j, carry):
                rv = rbuf[pl.ds(j * 16, 16)]
                ev = c * cchunk + j * 16 + iot
                return _scatter_vreg(rv, ev, None, carry)
            ptrs = lax.fori_loop(0, nfull, _part, ptrs)
            rv = rbuf[pl.ds(nfull * 16, 16)]
            ev = c * cchunk + nfull * 16 + iot
            ptrs = _scatter_vreg(rv, ev, iot < rem, ptrs)

        for b in range(NB):
            for z in range(rows_pt // zrows):
                pltpu.sync_copy(
                    zbuf, acc.at[pl.ds(sid * rows_pt + z * zrows, zrows)])
            plsc.subcore_barrier()

            cnt_b = cnt_s[b]
            st_b = start_s[b]
            nch = (cnt_b + (K - 1)) // K

            def _chunk(j, carry):
                cb = st_b + j * K
                for g in range(K // 16):
                    pv = plist[pl.ds(cb + 16 * g, 16)]
                    eidc[pl.ds(16 * g, 16)] = ebase + lax.shift_right_logical(pv, 13)
                    lrecc[pl.ds(16 * g, 16)] = pv & 8191
                pltpu.async_copy(edata.at[eidc], ebuf, sem).wait()
                for g in range(K // 16):
                    rows = 16 * g + iot
                    cols = zi16 + (ROW - 5)
                    sf = plsc.load_gather(ebuf, [rows, cols])
                    sidx[pl.ds(16 * g, 16)] = sf.astype(jnp.int32)
                pltpu.async_copy(ndata.at[sidx], nbuf, sem).wait()

                def _edge(ee, c2):
                    vmsk = jnp.where(j * K + ee < cnt_b, 1.0, 0.0)
                    w0 = ebuf[ee, 0:16] * vmsk
                    w1 = ebuf[ee, 16:32] * vmsk
                    w2 = ebuf[ee, 32:48] * vmsk
                    w3 = ebuf[ee, 48:64] * vmsk
                    yv = ebuf[ee, 56:72]
                    y10 = yv[8]
                    y11 = yv[9]
                    y12 = yv[10]
                    s = nbuf[ee, 0:16]
                    v0 = nbuf[ee, 16:32]
                    v1 = nbuf[ee, 32:48]
                    v2 = nbuf[ee, 48:64]
                    t = w1 * s
                    msgbuf[ee, 0:16] = w0 * s
                    msgbuf[ee, 16:32] = w3 * (v0 * y10 + v1 * y11 + v2 * y12)
                    msgbuf[ee, 32:48] = t * y10
                    msgbuf[ee, 48:64] = w2 * v0
                    msgbuf[ee, 64:80] = t * y11
                    msgbuf[ee, 80:96] = w2 * v1
                    msgbuf[ee, 96:112] = t * y12
                    msgbuf[ee, 112:128] = w2 * v2
                    return c2
                lax.fori_loop(0, K, _edge, 0)
                pltpu.sync_copy(msgbuf, acc.at[lrecc], add=True)
                return carry
            lax.fori_loop(0, nch, _chunk, 0)
            plsc.subcore_barrier()
            for z in range(rows_pt // zrows):
                r0 = sid * rows_pt + z * zrows
                pltpu.sync_copy(acc.at[pl.ds(r0, zrows)],
                                out.at[cid, pl.ds(b * rng + r0, zrows)])
            plsc.subcore_barrier()

    return body(edgedata, receivers, nodedata)


def _node_body(pa_ref, pb_ref, ns_ref, nv_ref, sp_ref,
               w0_ref, w1_ref, sk0_ref, sk1_ref, outs_ref, outv_ref):
    row = pa_ref[...] + pb_ref[...]
    cs = row[:, 0:32]
    sg = cs @ w0_ref[...]
    scal = ns_ref[...]
    spf = sp_ref[...].astype(jnp.float32)
    for k in range(4):
        mk = jnp.where(spf == float(k), 1.0, 0.0)
        sg = sg + mk * (scal @ sk0_ref[k])
    gates = jax.nn.sigmoid(sg[:, MUL:2 * MUL])
    s_out = sg[:, 0:MUL]
    outs_ref[...] = s_out * jax.nn.sigmoid(s_out)
    nv = nv_ref[...]
    for i in range(3):
        vi = row[:, 32 + 32 * i:64 + 32 * i] @ w1_ref[...]
        for k in range(4):
            mk = jnp.where(spf == float(k), 1.0, 0.0)
            vi = vi + mk * (nv[:, 16 * i:16 * i + 16] @ sk1_ref[k])
        outv_ref[:, 16 * i:16 * i + 16] = vi * gates


def _node_stage(pa, pb, node_scalars, nv48, species,
                w0s, w1s, sk0s, sk1s, block_n):
    n = node_scalars.shape[0]
    grid = n // block_n
    return pl.pallas_call(
        _node_body,
        grid=(grid,),
        in_specs=[
            pl.BlockSpec((block_n, 128), lambda i: (i, 0)),
            pl.BlockSpec((block_n, 128), lambda i: (i, 0)),
            pl.BlockSpec((block_n, 16), lambda i: (i, 0)),
            pl.BlockSpec((block_n, 48), lambda i: (i, 0)),
            pl.BlockSpec((block_n, 1), lambda i: (i, 0)),
            pl.BlockSpec((32, 32), lambda i: (0, 0)),
            pl.BlockSpec((32, 16), lambda i: (0, 0)),
            pl.BlockSpec((4, 16, 32), lambda i: (0, 0, 0)),
            pl.BlockSpec((4, 16, 16), lambda i: (0, 0, 0)),
        ],
        out_specs=[
            pl.BlockSpec((block_n, 16), lambda i: (i, 0)),
            pl.BlockSpec((block_n, 48), lambda i: (i, 0)),
        ],
        out_shape=[
            jax.ShapeDtypeStruct((n, 16), jnp.float32),
            jax.ShapeDtypeStruct((n, 48), jnp.float32),
        ],
    )(pa, pb, node_scalars, nv48, species.reshape(n, 1),
      w0s, w1s, sk0s, sk1s)


def _pick_block(total, want):
    b = min(want, total)
    while total % b:
        b -= 1
    return b


def kernel(node_scalars, node_vectors, vectors, species, senders, receivers,
           mlp_w1, mlp_w2, mlp_w3, lin_w0e, lin_w1o, skip_w0e, skip_w1o):
    e = vectors.shape[0]
    n = node_scalars.shape[0]

    isq3 = 1.0 / np.sqrt(3.0)
    path_scale = np.repeat(np.array([1.0, isq3, 1.0, isq3], np.float32), MUL)
    w3s = mlp_w3 * (EPS * path_scale)[None, :]

    block_e = _pick_block(e, 2000)
    edgedata = _edge_stage(vectors, senders, mlp_w1, mlp_w2, w3s, block_e)

    nv48 = node_vectors.transpose(0, 2, 1).reshape(n, 48)
    nodedata = jnp.concatenate([node_scalars, nv48], axis=1)

    partials = _sc_conv(edgedata, receivers, nodedata)

    rs2 = np.float32(1.0 / np.sqrt(2.0 * MUL))
    rs1 = np.float32(1.0 / np.sqrt(1.0 * MUL))
    block_n = _pick_block(n, 2000)
    out_s, out_v48 = _node_stage(
        partials[0], partials[1], node_scalars, nv48, species,
        lin_w0e * rs2, lin_w1o * rs2, skip_w0e * rs1, skip_w1o * rs1,
        block_n)
    out_v = out_v48.reshape(n, 3, MUL).transpose(0, 2, 1)
    return out_s, out_v


# speedup vs baseline: 42.1151x; 1.8492x over previous
import functools

import jax
import jax.numpy as jnp
import numpy as np
from jax import lax
from jax.experimental import pallas as pl
from jax.experimental.pallas import tpu as pltpu
from jax.experimental.pallas import tpu_sc as plsc

MUL = 16
RADIAL = 8
CUTOFF = 5.0
EPS = 0.25

NB = 8
K = 128
ROW = 72
NDROW = 64


def _edge_body(v_ref, snd_ref, w1t_ref, w2t_ref, w3t_ref, out_ref):
    v = v_ref[...]
    vt = v.T
    x = vt[0:1, :]
    y = vt[1:2, :]
    z = vt[2:3, :]
    r2 = x * x + y * y + z * z
    r = jnp.sqrt(r2)
    u = r * (1.0 / CUTOFF)
    u2 = u * u
    u4 = u2 * u2
    up = u4 * u
    env = 1.0 - 21.0 * up + 35.0 * up * u - 15.0 * up * u2
    env = jnp.where(u < 1.0, env, 0.0)
    px = jnp.pi * u
    s1 = jnp.sin(px)
    c1 = jnp.cos(px)
    twoc = c1 + c1
    ss = [s1, twoc * s1]
    for _ in range(6):
        ss.append(twoc * ss[-1] - ss[-2])
    sn = jnp.concatenate(ss, axis=0)
    pos = u > 0.0
    uinv = jnp.where(pos, 1.0 / jnp.where(pos, u, 1.0), 0.0)
    c0 = np.float32(np.sqrt(2.0 / CUTOFF) / CUTOFF)
    mult = c0 * env * uinv
    nsc = lax.broadcasted_iota(jnp.int32, (RADIAL, 1), 0).astype(jnp.float32)
    nsc = (nsc + 1.0) * np.float32(np.pi * np.sqrt(2.0 / CUTOFF) / CUTOFF)
    rbf = jnp.where(pos, sn * mult, nsc * env)
    h = w1t_ref[...] @ rbf
    h = h * jax.nn.sigmoid(h)
    h = w2t_ref[...] @ h
    h = h * jax.nn.sigmoid(h)
    tpw = w3t_ref[...] @ h
    rinv = jnp.sqrt(3.0) / (r + 1e-12)
    y1 = vt * rinv
    snd = snd_ref[...].astype(jnp.float32)
    pad = jnp.zeros_like(snd)
    rec = jnp.concatenate([tpw, y1, snd, pad, pad, pad, pad], axis=0)
    out_ref[...] = rec.T


def _edge_stage(vectors, senders, mlp_w1, mlp_w2, w3s, block_e):
    e = vectors.shape[0]
    grid = e // block_e
    return pl.pallas_call(
        _edge_body,
        grid=(grid,),
        in_specs=[
            pl.BlockSpec((block_e, 3), lambda i: (i, 0)),
            pl.BlockSpec((1, block_e), lambda i: (0, i)),
            pl.BlockSpec((64, RADIAL), lambda i: (0, 0)),
            pl.BlockSpec((64, 64), lambda i: (0, 0)),
            pl.BlockSpec((64, 64), lambda i: (0, 0)),
        ],
        out_specs=pl.BlockSpec((block_e, ROW), lambda i: (i, 0)),
        out_shape=jax.ShapeDtypeStruct((e, ROW), jnp.float32),
    )(vectors, senders.reshape(1, e), mlp_w1.T, mlp_w2.T, w3s.T)


def _sc_conv(edgedata, receivers, nodedata):
    e = edgedata.shape[0]
    n = nodedata.shape[0]
    info = plsc.get_sparse_core_info()
    nc, ns = info.num_cores, info.num_subcores
    nw = nc * ns
    epw = e // nw
    assert epw * nw == e
    quant = NB * ns * 8
    npad = ((n + quant - 1) // quant) * quant
    rng = npad // NB
    rows_pt = rng // ns
    zrows = max(z for z in range(8, 129, 8) if rows_pt % z == 0)
    assert rows_pt % zrows == 0 and zrows % 8 == 0
    cchunk = 5000
    assert epw % cchunk == 0 and cchunk % 8 == 0
    nchunks = epw // cchunk
    nfull = cchunk // 16
    rem = cchunk - nfull * 16
    cap = ((epw + NB * 16 + K + 15) // 16) * 16

    mesh = plsc.VectorSubcoreMesh(core_axis_name="c", subcore_axis_name="s")

    @functools.partial(
        pl.kernel,
        out_type=jax.ShapeDtypeStruct((nc, npad, 128), jnp.float32),
        mesh=mesh,
        compiler_params=pltpu.CompilerParams(needs_layout_passes=False, use_tc_tiling_on_sc=False),
        scratch_types=[
            pltpu.VMEM((cchunk + 16,), jnp.int32),
            pltpu.VMEM((cap,), jnp.int32),
            pltpu.VMEM((K,), jnp.int32),
            pltpu.VMEM((K,), jnp.int32),
            pltpu.VMEM((K,), jnp.int32),
            pltpu.VMEM((K, ROW), jnp.float32),
            pltpu.VMEM((K, NDROW), jnp.float32),
            pltpu.VMEM((K, 128), jnp.float32),
            pltpu.VMEM((zrows, 128), jnp.float32),
            pltpu.VMEM_SHARED((rng, 128), jnp.float32),
            pltpu.SemaphoreType.DMA,
        ],
    )
    def body(edata, recv, ndata, out, rbuf, plist, eidc, lrecc, sidx,
             ebuf, nbuf, msgbuf, zbuf, acc, sem):
        cid = lax.axis_index("c")
        sid = lax.axis_index("s")
        wid = cid * ns + sid
        ebase = wid * epw
        iot = lax.broadcasted_iota(jnp.int32, (16,), 0)
        zero16 = jnp.zeros((16,), jnp.float32)
        zi16 = jnp.zeros((16,), jnp.int32)

        def _zrow(r, c):
            for g in range(8):
                zbuf[r, 16 * g:16 * g + 16] = zero16
            return c
        lax.fori_loop(0, zrows, _zrow, 0)

        def _zlist(r, c):
            plist[pl.ds(r * 16, 16)] = zi16
            return c
        lax.fori_loop(0, cap // 16, _zlist, 0)

        cnts = tuple(jnp.int32(0) for _ in range(NB))
        for c in range(nchunks):
            pltpu.sync_copy(recv.at[pl.ds(ebase + c * cchunk, cchunk)],
                            rbuf.at[pl.ds(0, cchunk)])

            def _cnt(j, carry):
                rv = rbuf[pl.ds(j * 16, 16)]
                res = []
                for b in range(NB):
                    m = (rv >= b * rng) & (rv < (b + 1) * rng)
                    res.append(carry[b] + jnp.sum(m.astype(jnp.int32)))
                return tuple(res)
            cnts = lax.fori_loop(0, nfull, _cnt, cnts)
            rv = rbuf[pl.ds(nfull * 16, 16)]
            mv = iot < rem
            cnts = tuple(
                cnts[b] + jnp.sum(
                    ((rv >= b * rng) & (rv < (b + 1) * rng) & mv)
                    .astype(jnp.int32))
                for b in range(NB))

        cnt_s = list(cnts)
        start_s = []
        cur = jnp.int32(0)
        for b in range(NB):
            start_s.append(cur)
            cur = lax.bitwise_and(cur + cnt_s[b] + 15, jnp.int32(-16))

        ptrs = tuple(jnp.full((16,), start_s[b], jnp.int32) for b in range(NB))
        for c in range(nchunks):
            pltpu.sync_copy(recv.at[pl.ds(ebase + c * cchunk, cchunk)],
                            rbuf.at[pl.ds(0, cchunk)])

            def _scatter_vreg(rv, ev, extra_mask, ptr):
                res = []
                for b in range(NB):
                    m = (rv >= b * rng) & (rv < (b + 1) * rng)
                    if extra_mask is not None:
                        m = m & extra_mask
                    ranks = plsc.cumsum(m.astype(jnp.int32))
                    pos = ptr[b] + ranks - 1
                    packed = lax.shift_left(ev, 13) | (rv - b * rng)
                    plsc.store_scatter(plist, [pos], packed, mask=m)
                    res.append(ptr[b] + jnp.sum(m.astype(jnp.int32)))
                return tuple(res)

            def _part(j, carry):
                rv = rbuf[pl.ds(j * 16, 16)]
                ev = c * cchunk + j * 16 + iot
                return _scatter_vreg(rv, ev, None, carry)
            ptrs = lax.fori_loop(0, nfull, _part, ptrs)
            rv = rbuf[pl.ds(nfull * 16, 16)]
            ev = c * cchunk + nfull * 16 + iot
            ptrs = _scatter_vreg(rv, ev, iot < rem, ptrs)

        for b in range(NB):
            for z in range(rows_pt // zrows):
                pltpu.sync_copy(
                    zbuf, acc.at[pl.ds(sid * rows_pt + z * zrows, zrows)])
            plsc.subcore_barrier()

            cnt_b = cnt_s[b]
            st_b = start_s[b]
            nch = (cnt_b + (K - 1)) // K

            def _chunk(j, carry):
                cb = st_b + j * K
                for g in range(K // 16):
                    pv = plist[pl.ds(cb + 16 * g, 16)]
                    eidc[pl.ds(16 * g, 16)] = ebase + lax.shift_right_logical(pv, 13)
                    lrecc[pl.ds(16 * g, 16)] = pv & 8191
                pltpu.async_copy(edata.at[eidc], ebuf, sem).wait()
                for g in range(K // 16):
                    rows = 16 * g + iot
                    cols = zi16 + (ROW - 5)
                    sf = plsc.load_gather(ebuf, [rows, cols])
                    sidx[pl.ds(16 * g, 16)] = sf.astype(jnp.int32)
                pltpu.async_copy(ndata.at[sidx], nbuf, sem).wait()

                def _edge(ee, c2):
                    vmsk = jnp.where(j * K + ee < cnt_b, 1.0, 0.0)
                    w0 = ebuf[ee, 0:16] * vmsk
                    w1 = ebuf[ee, 16:32] * vmsk
                    w2 = ebuf[ee, 32:48] * vmsk
                    w3 = ebuf[ee, 48:64] * vmsk
                    yv = ebuf[ee, 56:72]
                    y10 = yv[8]
                    y11 = yv[9]
                    y12 = yv[10]
                    s = nbuf[ee, 0:16]
                    v0 = nbuf[ee, 16:32]
                    v1 = nbuf[ee, 32:48]
                    v2 = nbuf[ee, 48:64]
                    t = w1 * s
                    msgbuf[ee, 0:16] = w0 * s
                    msgbuf[ee, 16:32] = w3 * (v0 * y10 + v1 * y11 + v2 * y12)
                    msgbuf[ee, 32:48] = t * y10
                    msgbuf[ee, 48:64] = w2 * v0
                    msgbuf[ee, 64:80] = t * y11
                    msgbuf[ee, 80:96] = w2 * v1
                    msgbuf[ee, 96:112] = t * y12
                    msgbuf[ee, 112:128] = w2 * v2
                    return c2
                lax.fori_loop(0, K, _edge, 0)
                pltpu.sync_copy(msgbuf, acc.at[lrecc], add=True)
                return carry
            lax.fori_loop(0, nch, _chunk, 0)
            plsc.subcore_barrier()
            for z in range(rows_pt // zrows):
                r0 = sid * rows_pt + z * zrows
                pltpu.sync_copy(acc.at[pl.ds(r0, zrows)],
                                out.at[cid, pl.ds(b * rng + r0, zrows)])
            plsc.subcore_barrier()

    return body(edgedata, receivers, nodedata)


def _prep_body(ns_ref, nvm_ref, p48_ref, out_ref):
    out_ref[:, 0:16] = ns_ref[...]
    out_ref[:, 16:64] = nvm_ref[...] @ p48_ref[...]


def _prep_stage(node_scalars, nvm48, block_n):
    n = node_scalars.shape[0]
    p48 = np.zeros((48, 48), np.float32)
    for m in range(16):
        for i in range(3):
            p48[m * 3 + i, i * 16 + m] = 1.0
    grid = n // block_n
    return pl.pallas_call(
        _prep_body,
        grid=(grid,),
        in_specs=[
            pl.BlockSpec((block_n, 16), lambda i: (i, 0)),
            pl.BlockSpec((block_n, 48), lambda i: (i, 0)),
            pl.BlockSpec((48, 48), lambda i: (0, 0)),
        ],
        out_specs=pl.BlockSpec((block_n, 64), lambda i: (i, 0)),
        out_shape=jax.ShapeDtypeStruct((n, 64), jnp.float32),
    )(node_scalars, nvm48, p48)


def _node_body(pa_ref, pb_ref, ns_ref, nvm_ref, sp_ref,
               w0_ref, w1_ref, sk0_ref, skv_ref, pp_ref, outs_ref, outv_ref):
    row = pa_ref[...] + pb_ref[...]
    cs = row[:, 0:32]
    sg = cs @ w0_ref[...]
    scal = ns_ref[...]
    spf = sp_ref[...].astype(jnp.float32)
    masks = [jnp.where(spf == float(k), 1.0, 0.0) for k in range(4)]
    for k in range(4):
        sg = sg + masks[k] * (scal @ sk0_ref[k])
    gates = jax.nn.sigmoid(sg[:, MUL:2 * MUL])
    s_out = sg[:, 0:MUL]
    outs_ref[...] = s_out * jax.nn.sigmoid(s_out)
    nvm = nvm_ref[...]
    outm = jnp.zeros_like(nvm)
    for i in range(3):
        vi = row[:, 32 + 32 * i:64 + 32 * i] @ w1_ref[...]
        for k in range(4):
            vi = vi + masks[k] * (nvm @ skv_ref[k * 3 + i])
        outm = outm + (vi * gates) @ pp_ref[i]
    outv_ref[...] = outm


def _node_stage(pa, pb, node_scalars, nvm48, species,
                w0s, w1s, sk0s, sk1s, block_n):
    n = node_scalars.shape[0]
    m3 = np.zeros((3, 16, 48), np.float32)
    for i in range(3):
        for m in range(16):
            m3[i, m, m * 3 + i] = 1.0
    skv = jnp.einsum('imp,kmo->kipo', m3, sk1s).reshape(12, 48, 16)
    pp = jnp.asarray(m3)
    grid = n // block_n
    return pl.pallas_call(
        _node_body,
        grid=(grid,),
        in_specs=[
            pl.BlockSpec((block_n, 128), lambda i: (i, 0)),
            pl.BlockSpec((block_n, 128), lambda i: (i, 0)),
            pl.BlockSpec((block_n, 16), lambda i: (i, 0)),
            pl.BlockSpec((block_n, 48), lambda i: (i, 0)),
            pl.BlockSpec((block_n, 1), lambda i: (i, 0)),
            pl.BlockSpec((32, 32), lambda i: (0, 0)),
            pl.BlockSpec((32, 16), lambda i: (0, 0)),
            pl.BlockSpec((4, 16, 32), lambda i: (0, 0, 0)),
            pl.BlockSpec((12, 48, 16), lambda i: (0, 0, 0)),
            pl.BlockSpec((3, 16, 48), lambda i: (0, 0, 0)),
        ],
        out_specs=[
            pl.BlockSpec((block_n, 16), lambda i: (i, 0)),
            pl.BlockSpec((block_n, 48), lambda i: (i, 0)),
        ],
        out_shape=[
            jax.ShapeDtypeStruct((n, 16), jnp.float32),
            jax.ShapeDtypeStruct((n, 48), jnp.float32),
        ],
    )(pa, pb, node_scalars, nvm48, species.reshape(n, 1),
      w0s, w1s, sk0s, skv, pp)


def _pick_block(total, want):
    b = min(want, total)
    while total % b:
        b -= 1
    return b


def kernel(node_scalars, node_vectors, vectors, species, senders, receivers,
           mlp_w1, mlp_w2, mlp_w3, lin_w0e, lin_w1o, skip_w0e, skip_w1o):
    e = vectors.shape[0]
    n = node_scalars.shape[0]

    isq3 = 1.0 / np.sqrt(3.0)
    path_scale = np.repeat(np.array([1.0, isq3, 1.0, isq3], np.float32), MUL)
    w3s = mlp_w3 * (EPS * path_scale)[None, :]

    block_e = 3200 if e % 3200 == 0 else e
    edgedata = _edge_stage(vectors, senders, mlp_w1, mlp_w2, w3s, block_e)

    nvm48 = node_vectors.reshape(n, 48)
    block_n = _pick_block(n, 2000)
    nodedata = _prep_stage(node_scalars, nvm48, block_n)

    partials = _sc_conv(edgedata, receivers, nodedata)

    rs2 = np.float32(1.0 / np.sqrt(2.0 * MUL))
    rs1 = np.float32(1.0 / np.sqrt(1.0 * MUL))
    out_s, out_v48m = _node_stage(
        partials[0], partials[1], node_scalars, nvm48, species,
        lin_w0e * rs2, lin_w1o * rs2, skip_w0e * rs1, skip_w1o * rs1,
        block_n)
    out_v = out_v48m.reshape(n, MUL, 3)
    return out_s, out_v


# speedup vs baseline: 56.1578x; 1.3334x over previous
import functools

import jax
import jax.numpy as jnp
import numpy as np
from jax import lax
from jax.experimental import pallas as pl
from jax.experimental.pallas import tpu as pltpu
from jax.experimental.pallas import tpu_sc as plsc

MUL = 16
RADIAL = 8
CUTOFF = 5.0
EPS = 0.25

NB = 8
K = 64
ROW = 72
NDROW = 64


def _edge_body(v_ref, snd_ref, w1t_ref, w2t_ref, w3t_ref, out_ref):
    v = v_ref[...]
    vt = v.T
    x = vt[0:1, :]
    y = vt[1:2, :]
    z = vt[2:3, :]
    r2 = x * x + y * y + z * z
    r = jnp.sqrt(r2)
    u = r * (1.0 / CUTOFF)
    u2 = u * u
    u4 = u2 * u2
    up = u4 * u
    env = 1.0 - 21.0 * up + 35.0 * up * u - 15.0 * up * u2
    env = jnp.where(u < 1.0, env, 0.0)
    px = jnp.pi * u
    s1 = jnp.sin(px)
    c1 = jnp.cos(px)
    twoc = c1 + c1
    ss = [s1, twoc * s1]
    for _ in range(6):
        ss.append(twoc * ss[-1] - ss[-2])
    sn = jnp.concatenate(ss, axis=0)
    pos = u > 0.0
    uinv = jnp.where(pos, 1.0 / jnp.where(pos, u, 1.0), 0.0)
    c0 = np.float32(np.sqrt(2.0 / CUTOFF) / CUTOFF)
    mult = c0 * env * uinv
    nsc = lax.broadcasted_iota(jnp.int32, (RADIAL, 1), 0).astype(jnp.float32)
    nsc = (nsc + 1.0) * np.float32(np.pi * np.sqrt(2.0 / CUTOFF) / CUTOFF)
    rbf = jnp.where(pos, sn * mult, nsc * env)
    h = w1t_ref[...] @ rbf
    h = h * jax.nn.sigmoid(h)
    h = w2t_ref[...] @ h
    h = h * jax.nn.sigmoid(h)
    tpw = w3t_ref[...] @ h
    rinv = jnp.sqrt(3.0) / (r + 1e-12)
    y1 = vt * rinv
    snd = snd_ref[...].astype(jnp.float32)
    pad = jnp.zeros_like(snd)
    rec = jnp.concatenate([tpw, y1, snd, pad, pad, pad, pad], axis=0)
    out_ref[...] = rec.T


def _edge_stage(vectors, senders, mlp_w1, mlp_w2, w3s, block_e):
    e = vectors.shape[0]
    grid = e // block_e
    return pl.pallas_call(
        _edge_body,
        grid=(grid,),
        in_specs=[
            pl.BlockSpec((block_e, 3), lambda i: (i, 0)),
            pl.BlockSpec((1, block_e), lambda i: (0, i)),
            pl.BlockSpec((64, RADIAL), lambda i: (0, 0)),
            pl.BlockSpec((64, 64), lambda i: (0, 0)),
            pl.BlockSpec((64, 64), lambda i: (0, 0)),
        ],
        out_specs=pl.BlockSpec((block_e, ROW), lambda i: (i, 0)),
        out_shape=jax.ShapeDtypeStruct((e, ROW), jnp.float32),
    )(vectors, senders.reshape(1, e), mlp_w1.T, mlp_w2.T, w3s.T)


def _sc_conv(edgedata, receivers, nodedata):
    e = edgedata.shape[0]
    n = nodedata.shape[0]
    info = plsc.get_sparse_core_info()
    nc, ns = info.num_cores, info.num_subcores
    nw = nc * ns
    epw = e // nw
    assert epw * nw == e
    quant = NB * ns * 8
    npad = ((n + quant - 1) // quant) * quant
    rng = npad // NB
    rows_pt = rng // ns
    zrows = max(z for z in range(8, 129, 8) if rows_pt % z == 0)
    assert rows_pt % zrows == 0 and zrows % 8 == 0
    cchunk = 5000
    assert epw % cchunk == 0 and cchunk % 8 == 0
    nchunks = epw // cchunk
    nfull = cchunk // 16
    rem = cchunk - nfull * 16
    cap = ((epw + NB * 16 + K + 15) // 16) * 16

    mesh = plsc.VectorSubcoreMesh(core_axis_name="c", subcore_axis_name="s")

    @functools.partial(
        pl.kernel,
        out_type=jax.ShapeDtypeStruct((nc, npad, 128), jnp.float32),
        mesh=mesh,
        compiler_params=pltpu.CompilerParams(needs_layout_passes=False, use_tc_tiling_on_sc=False),
        scratch_types=[
            pltpu.VMEM((cchunk + 16,), jnp.int32),
            pltpu.VMEM((cap,), jnp.int32),
            pltpu.VMEM((K,), jnp.int32),
            pltpu.VMEM((K,), jnp.int32),
            pltpu.VMEM((K,), jnp.int32),
            pltpu.VMEM((K,), jnp.int32),
            pltpu.VMEM((K,), jnp.int32),
            pltpu.VMEM((K,), jnp.int32),
            pltpu.VMEM((K, ROW), jnp.float32),
            pltpu.VMEM((K, ROW), jnp.float32),
            pltpu.VMEM((K, NDROW), jnp.float32),
            pltpu.VMEM((K, NDROW), jnp.float32),
            pltpu.VMEM((K, 128), jnp.float32),
            pltpu.VMEM((K, 128), jnp.float32),
            pltpu.VMEM((zrows, 128), jnp.float32),
            pltpu.VMEM_SHARED((rng, 128), jnp.float32),
            pltpu.SemaphoreType.DMA,
            pltpu.SemaphoreType.DMA,
            pltpu.SemaphoreType.DMA,
            pltpu.SemaphoreType.DMA,
        ],
    )
    def body(edata, recv, ndata, out, rbuf, plist,
             eidc0, eidc1, lrecc0, lrecc1, sidx0, sidx1,
             ebuf0, ebuf1, nbuf0, nbuf1, msgbuf0, msgbuf1,
             zbuf, acc, seme0, seme1, semn0, semn1):
        cid = lax.axis_index("c")
        sid = lax.axis_index("s")
        wid = cid * ns + sid
        ebase = wid * epw
        iot = lax.broadcasted_iota(jnp.int32, (16,), 0)
        zero16 = jnp.zeros((16,), jnp.float32)
        zi16 = jnp.zeros((16,), jnp.int32)

        def _zrow(r, c):
            for g in range(8):
                zbuf[r, 16 * g:16 * g + 16] = zero16
            return c
        lax.fori_loop(0, zrows, _zrow, 0)

        def _zlist(r, c):
            plist[pl.ds(r * 16, 16)] = zi16
            return c
        lax.fori_loop(0, cap // 16, _zlist, 0)

        cnts = tuple(jnp.int32(0) for _ in range(NB))
        for c in range(nchunks):
            pltpu.sync_copy(recv.at[pl.ds(ebase + c * cchunk, cchunk)],
                            rbuf.at[pl.ds(0, cchunk)])

            def _cnt(j, carry):
                rv = rbuf[pl.ds(j * 16, 16)]
                res = []
                for b in range(NB):
                    m = (rv >= b * rng) & (rv < (b + 1) * rng)
                    res.append(carry[b] + jnp.sum(m.astype(jnp.int32)))
                return tuple(res)
            cnts = lax.fori_loop(0, nfull, _cnt, cnts)
            rv = rbuf[pl.ds(nfull * 16, 16)]
            mv = iot < rem
            cnts = tuple(
                cnts[b] + jnp.sum(
                    ((rv >= b * rng) & (rv < (b + 1) * rng) & mv)
                    .astype(jnp.int32))
                for b in range(NB))

        cnt_s = list(cnts)
        start_s = []
        cur = jnp.int32(0)
        for b in range(NB):
            start_s.append(cur)
            cur = lax.bitwise_and(cur + cnt_s[b] + 15, jnp.int32(-16))

        ptrs = tuple(jnp.full((16,), start_s[b], jnp.int32) for b in range(NB))
        for c in range(nchunks):
            pltpu.sync_copy(recv.at[pl.ds(ebase + c * cchunk, cchunk)],
                            rbuf.at[pl.ds(0, cchunk)])

            def _scatter_vreg(rv, ev, extra_mask, ptr):
                res = []
                for b in range(NB):
                    m = (rv >= b * rng) & (rv < (b + 1) * rng)
                    if extra_mask is not None:
                        m = m & extra_mask
                    ranks = plsc.cumsum(m.astype(jnp.int32))
                    pos = ptr[b] + ranks - 1
                    packed = lax.shift_left(ev, 13) | (rv - b * rng)
                    plsc.store_scatter(plist, [pos], packed, mask=m)
                    res.append(ptr[b] + jnp.sum(m.astype(jnp.int32)))
                return tuple(res)

            def _part(j, carry):
                rv = rbuf[pl.ds(j * 16, 16)]
                ev = c * cchunk + j * 16 + iot
                return _scatter_vreg(rv, ev, None, carry)
            ptrs = lax.fori_loop(0, nfull, _part, ptrs)
            rv = rbuf[pl.ds(nfull * 16, 16)]
            ev = c * cchunk + nfull * 16 + iot
            ptrs = _scatter_vreg(rv, ev, iot < rem, ptrs)

        eidc = [eidc0, eidc1]
        lrecc = [lrecc0, lrecc1]
        sidx = [sidx0, sidx1]
        ebuf = [ebuf0, ebuf1]
        nbuf = [nbuf0, nbuf1]
        msgbuf = [msgbuf0, msgbuf1]
        seme = [seme0, seme1]
        semn = [semn0, semn1]

        for b in range(NB):
            for z in range(rows_pt // zrows):
                pltpu.sync_copy(
                    zbuf, acc.at[pl.ds(sid * rows_pt + z * zrows, zrows)])
            plsc.subcore_barrier()

            cnt_b = cnt_s[b]
            st_b = start_s[b]
            nch = (cnt_b + (K - 1)) // K
            npairs = (nch + 1) // 2

            def _stage(j, p):
                cb = st_b + j * K
                for g in range(K // 16):
                    pv = plist[pl.ds(cb + 16 * g, 16)]
                    eidc[p][pl.ds(16 * g, 16)] = (
                        ebase + lax.shift_right_logical(pv, 13))
                    lrecc[p][pl.ds(16 * g, 16)] = pv & 8191
                return pltpu.async_copy(edata.at[eidc[p]], ebuf[p], seme[p])

            def _extract(j, p):
                for g in range(K // 16):
                    rows = 16 * g + iot
                    cols = zi16 + (ROW - 5)
                    sf = plsc.load_gather(ebuf[p], [rows, cols])
                    sidx[p][pl.ds(16 * g, 16)] = sf.astype(jnp.int32)
                return pltpu.async_copy(ndata.at[sidx[p]], nbuf[p], semn[p])

            def _compute(j, p):
                eb = ebuf[p]
                nb = nbuf[p]
                mb = msgbuf[p]

                def _edge(ee, c2):
                    vmsk = jnp.where(j * K + ee < cnt_b, 1.0, 0.0)
                    w0 = eb[ee, 0:16] * vmsk
                    w1 = eb[ee, 16:32] * vmsk
                    w2 = eb[ee, 32:48] * vmsk
                    w3 = eb[ee, 48:64] * vmsk
                    yv = eb[ee, 56:72]
                    y10 = yv[8]
                    y11 = yv[9]
                    y12 = yv[10]
                    sv = nb[ee, 0:16]
                    v0 = nb[ee, 16:32]
                    v1 = nb[ee, 32:48]
                    v2 = nb[ee, 48:64]
                    t = w1 * sv
                    mb[ee, 0:16] = w0 * sv
                    mb[ee, 16:32] = w3 * (v0 * y10 + v1 * y11 + v2 * y12)
                    mb[ee, 32:48] = t * y10
                    mb[ee, 48:64] = w2 * v0
                    mb[ee, 64:80] = t * y11
                    mb[ee, 80:96] = w2 * v1
                    mb[ee, 96:112] = t * y12
                    mb[ee, 112:128] = w2 * v2
                    return c2
                lax.fori_loop(0, K, _edge, 0)
                pltpu.sync_copy(mb, acc.at[lrecc[p]], add=True)

            @pl.when(nch > 0)
            def _prime0():
                _stage(jnp.int32(0), 0)

            @pl.when(nch > 1)
            def _prime1():
                _stage(jnp.int32(1), 1)

            def _pair(t, carry):
                j0 = 2 * t
                j1 = 2 * t + 1
                pltpu.make_async_copy(edata.at[eidc[0]], ebuf[0],
                                      seme[0]).wait()
                _extract(j0, 0)

                @pl.when(j1 < nch)
                def _x1():
                    pltpu.make_async_copy(edata.at[eidc[1]], ebuf[1],
                                          seme[1]).wait()
                    _extract(j1, 1)

                pltpu.make_async_copy(ndata.at[sidx[0]], nbuf[0],
                                      semn[0]).wait()
                _compute(j0, 0)

                @pl.when(j0 + 2 < nch)
                def _s2():
                    _stage(j0 + 2, 0)

                @pl.when(j1 < nch)
                def _c1():
                    pltpu.make_async_copy(ndata.at[sidx[1]], nbuf[1],
                                          semn[1]).wait()
                    _compute(j1, 1)

                    @pl.when(j1 + 2 < nch)
                    def _s3():
                        _stage(j1 + 2, 1)
                return carry
            lax.fori_loop(0, npairs, _pair, 0)
            plsc.subcore_barrier()
            for z in range(rows_pt // zrows):
                r0 = sid * rows_pt + z * zrows
                pltpu.sync_copy(acc.at[pl.ds(r0, zrows)],
                                out.at[cid, pl.ds(b * rng + r0, zrows)])
            plsc.subcore_barrier()

    return body(edgedata, receivers, nodedata)


def _prep_body(ns_ref, nvm_ref, p48_ref, out_ref):
    out_ref[:, 0:16] = ns_ref[...]
    out_ref[:, 16:64] = nvm_ref[...] @ p48_ref[...]


def _prep_stage(node_scalars, nvm48, block_n):
    n = node_scalars.shape[0]
    p48 = np.zeros((48, 48), np.float32)
    for m in range(16):
        for i in range(3):
            p48[m * 3 + i, i * 16 + m] = 1.0
    grid = n // block_n
    return pl.pallas_call(
        _prep_body,
        grid=(grid,),
        in_specs=[
            pl.BlockSpec((block_n, 16), lambda i: (i, 0)),
            pl.BlockSpec((block_n, 48), lambda i: (i, 0)),
            pl.BlockSpec((48, 48), lambda i: (0, 0)),
        ],
        out_specs=pl.BlockSpec((block_n, 64), lambda i: (i, 0)),
        out_shape=jax.ShapeDtypeStruct((n, 64), jnp.float32),
    )(node_scalars, nvm48, p48)


def _node_body(pa_ref, pb_ref, ns_ref, nvm_ref, sp_ref,
               w0_ref, w1_ref, sk0_ref, skv_ref, pp_ref, outs_ref, outv_ref):
    row = pa_ref[...] + pb_ref[...]
    cs = row[:, 0:32]
    sg = cs @ w0_ref[...]
    scal = ns_ref[...]
    spf = sp_ref[...].astype(jnp.float32)
    masks = [jnp.where(spf == float(k), 1.0, 0.0) for k in range(4)]
    for k in range(4):
        sg = sg + masks[k] * (scal @ sk0_ref[k])
    gates = jax.nn.sigmoid(sg[:, MUL:2 * MUL])
    s_out = sg[:, 0:MUL]
    outs_ref[...] = s_out * jax.nn.sigmoid(s_out)
    nvm = nvm_ref[...]
    outm = jnp.zeros_like(nvm)
    for i in range(3):
        vi = row[:, 32 + 32 * i:64 + 32 * i] @ w1_ref[...]
        for k in range(4):
            vi = vi + masks[k] * (nvm @ skv_ref[k * 3 + i])
        outm = outm + (vi * gates) @ pp_ref[i]
    outv_ref[...] = outm


def _node_stage(pa, pb, node_scalars, nvm48, species,
                w0s, w1s, sk0s, sk1s, block_n):
    n = node_scalars.shape[0]
    m3 = np.zeros((3, 16, 48), np.float32)
    for i in range(3):
        for m in range(16):
            m3[i, m, m * 3 + i] = 1.0
    skv = jnp.einsum('imp,kmo->kipo', m3, sk1s).reshape(12, 48, 16)
    pp = jnp.asarray(m3)
    grid = n // block_n
    return pl.pallas_call(
        _node_body,
        grid=(grid,),
        in_specs=[
            pl.BlockSpec((block_n, 128), lambda i: (i, 0)),
            pl.BlockSpec((block_n, 128), lambda i: (i, 0)),
            pl.BlockSpec((block_n, 16), lambda i: (i, 0)),
            pl.BlockSpec((block_n, 48), lambda i: (i, 0)),
            pl.BlockSpec((block_n, 1), lambda i: (i, 0)),
            pl.BlockSpec((32, 32), lambda i: (0, 0)),
            pl.BlockSpec((32, 16), lambda i: (0, 0)),
            pl.BlockSpec((4, 16, 32), lambda i: (0, 0, 0)),
            pl.BlockSpec((12, 48, 16), lambda i: (0, 0, 0)),
            pl.BlockSpec((3, 16, 48), lambda i: (0, 0, 0)),
        ],
        out_specs=[
            pl.BlockSpec((block_n, 16), lambda i: (i, 0)),
            pl.BlockSpec((block_n, 48), lambda i: (i, 0)),
        ],
        out_shape=[
            jax.ShapeDtypeStruct((n, 16), jnp.float32),
            jax.ShapeDtypeStruct((n, 48), jnp.float32),
        ],
    )(pa, pb, node_scalars, nvm48, species.reshape(n, 1),
      w0s, w1s, sk0s, skv, pp)


def _pick_block(total, want):
    b = min(want, total)
    while total % b:
        b -= 1
    return b


def kernel(node_scalars, node_vectors, vectors, species, senders, receivers,
           mlp_w1, mlp_w2, mlp_w3, lin_w0e, lin_w1o, skip_w0e, skip_w1o):
    e = vectors.shape[0]
    n = node_scalars.shape[0]

    isq3 = 1.0 / np.sqrt(3.0)
    path_scale = np.repeat(np.array([1.0, isq3, 1.0, isq3], np.float32), MUL)
    w3s = mlp_w3 * (EPS * path_scale)[None, :]

    block_e = 3200 if e % 3200 == 0 else e
    edgedata = _edge_stage(vectors, senders, mlp_w1, mlp_w2, w3s, block_e)

    nvm48 = node_vectors.reshape(n, 48)
    block_n = _pick_block(n, 2000)
    nodedata = _prep_stage(node_scalars, nvm48, block_n)

    partials = _sc_conv(edgedata, receivers, nodedata)

    rs2 = np.float32(1.0 / np.sqrt(2.0 * MUL))
    rs1 = np.float32(1.0 / np.sqrt(1.0 * MUL))
    out_s, out_v48m = _node_stage(
        partials[0], partials[1], node_scalars, nvm48, species,
        lin_w0e * rs2, lin_w1o * rs2, skip_w0e * rs1, skip_w1o * rs1,
        block_n)
    out_v = out_v48m.reshape(n, MUL, 3)
    return out_s, out_v


# speedup vs baseline: 63.7825x; 1.1358x over previous
import functools

import jax
import jax.numpy as jnp
import numpy as np
from jax import lax
from jax.experimental import pallas as pl
from jax.experimental.pallas import tpu as pltpu
from jax.experimental.pallas import tpu_sc as plsc

MUL = 16
RADIAL = 8
CUTOFF = 5.0
EPS = 0.25

NB = 8
K = 128
ROW = 128
NDROW = 64


def _edge_body(v_ref, snd_ref, w1t_ref, w2t_ref, w3t_ref, out_ref):
    vt = v_ref[...]
    x = vt[0:1, :]
    y = vt[1:2, :]
    z = vt[2:3, :]
    r2 = x * x + y * y + z * z
    r = jnp.sqrt(r2)
    u = r * (1.0 / CUTOFF)
    u2 = u * u
    u4 = u2 * u2
    up = u4 * u
    env = 1.0 - 21.0 * up + 35.0 * up * u - 15.0 * up * u2
    env = jnp.where(u < 1.0, env, 0.0)
    px = jnp.pi * u
    s1 = jnp.sin(px)
    c1 = jnp.cos(px)
    twoc = c1 + c1
    ss = [s1, twoc * s1]
    for _ in range(6):
        ss.append(twoc * ss[-1] - ss[-2])
    sn = jnp.concatenate(ss, axis=0)
    pos = u > 0.0
    uinv = jnp.where(pos, 1.0 / jnp.where(pos, u, 1.0), 0.0)
    c0 = np.float32(np.sqrt(2.0 / CUTOFF) / CUTOFF)
    mult = c0 * env * uinv
    nsc = lax.broadcasted_iota(jnp.int32, (RADIAL, 1), 0).astype(jnp.float32)
    nsc = (nsc + 1.0) * np.float32(np.pi * np.sqrt(2.0 / CUTOFF) / CUTOFF)
    rbf = jnp.where(pos, sn * mult, nsc * env)
    h = w1t_ref[...] @ rbf
    h = h * jax.nn.sigmoid(h)
    h = w2t_ref[...] @ h
    h = h * jax.nn.sigmoid(h)
    tpw = w3t_ref[...] @ h
    rinv = jnp.sqrt(3.0) / (r + 1e-12)
    y1 = vt * rinv
    snd = snd_ref[...].astype(jnp.float32)
    rec = jnp.concatenate([tpw, y1, snd], axis=0)
    out_ref[:, 0:68] = rec.T


def _edge_stage(vectors, senders, mlp_w1, mlp_w2, w3s, block_e):
    e = vectors.shape[0]
    grid = e // block_e
    return pl.pallas_call(
        _edge_body,
        grid=(grid,),
        in_specs=[
            pl.BlockSpec((3, block_e), lambda i: (0, i)),
            pl.BlockSpec((1, block_e), lambda i: (0, i)),
            pl.BlockSpec((64, RADIAL), lambda i: (0, 0)),
            pl.BlockSpec((64, 64), lambda i: (0, 0)),
            pl.BlockSpec((64, 64), lambda i: (0, 0)),
        ],
        out_specs=pl.BlockSpec((block_e, ROW), lambda i: (i, 0)),
        out_shape=jax.ShapeDtypeStruct((e, ROW), jnp.float32),
    )(vectors.T, senders.reshape(1, e), mlp_w1.T, mlp_w2.T, w3s.T)


def _sc_conv(edgedata, receivers, nodedata):
    e = edgedata.shape[0]
    n = nodedata.shape[0]
    info = plsc.get_sparse_core_info()
    nc, ns = info.num_cores, info.num_subcores
    nw = nc * ns
    epw = e // nw
    assert epw * nw == e
    quant = NB * ns * 8
    npad = ((n + quant - 1) // quant) * quant
    rng = npad // NB
    rows_pt = rng // ns
    zrows = max(z for z in range(8, 129, 8) if rows_pt % z == 0)
    assert rows_pt % zrows == 0 and zrows % 8 == 0
    cchunk = 5000
    assert epw % cchunk == 0 and cchunk % 8 == 0
    nchunks = epw // cchunk
    nfull = cchunk // 16
    rem = cchunk - nfull * 16
    cap = ((epw + NB * 16 + K + 15) // 16) * 16

    mesh = plsc.VectorSubcoreMesh(core_axis_name="c", subcore_axis_name="s")

    @functools.partial(
        pl.kernel,
        out_type=jax.ShapeDtypeStruct((nc, npad, 128), jnp.float32),
        mesh=mesh,
        compiler_params=pltpu.CompilerParams(needs_layout_passes=False, use_tc_tiling_on_sc=False),
        scratch_types=[
            pltpu.VMEM((cchunk + 16,), jnp.int32),
            pltpu.VMEM((cap,), jnp.int32),
            pltpu.VMEM((K,), jnp.int32),
            pltpu.VMEM((K,), jnp.int32),
            pltpu.VMEM((K,), jnp.int32),
            pltpu.VMEM((K, ROW), jnp.float32),
            pltpu.VMEM((K, NDROW), jnp.float32),
            pltpu.VMEM((K, 128), jnp.float32),
            pltpu.VMEM((zrows, 128), jnp.float32),
            pltpu.VMEM_SHARED((rng, 128), jnp.float32),
            pltpu.SemaphoreType.DMA,
        ],
    )
    def body(edata, recv, ndata, out, rbuf, plist, eidc, lrecc, sidx,
             ebuf, nbuf, msgbuf, zbuf, acc, sem):
        cid = lax.axis_index("c")
        sid = lax.axis_index("s")
        wid = cid * ns + sid
        ebase = wid * epw
        iot = lax.broadcasted_iota(jnp.int32, (16,), 0)
        zero16 = jnp.zeros((16,), jnp.float32)
        zi16 = jnp.zeros((16,), jnp.int32)

        def _zrow(r, c):
            for g in range(8):
                zbuf[r, 16 * g:16 * g + 16] = zero16
            return c
        lax.fori_loop(0, zrows, _zrow, 0)

        def _zlist(r, c):
            plist[pl.ds(r * 16, 16)] = zi16
            return c
        lax.fori_loop(0, cap // 16, _zlist, 0)

        cnts = tuple(jnp.int32(0) for _ in range(NB))
        for c in range(nchunks):
            pltpu.sync_copy(recv.at[pl.ds(ebase + c * cchunk, cchunk)],
                            rbuf.at[pl.ds(0, cchunk)])

            def _cnt(j, carry):
                rv = rbuf[pl.ds(j * 16, 16)]
                res = []
                for b in range(NB):
                    m = (rv >= b * rng) & (rv < (b + 1) * rng)
                    res.append(carry[b] + jnp.sum(m.astype(jnp.int32)))
                return tuple(res)
            cnts = lax.fori_loop(0, nfull, _cnt, cnts)
            rv = rbuf[pl.ds(nfull * 16, 16)]
            mv = iot < rem
            cnts = tuple(
                cnts[b] + jnp.sum(
                    ((rv >= b * rng) & (rv < (b + 1) * rng) & mv)
                    .astype(jnp.int32))
                for b in range(NB))

        cnt_s = list(cnts)
        start_s = []
        cur = jnp.int32(0)
        for b in range(NB):
            start_s.append(cur)
            cur = lax.bitwise_and(cur + cnt_s[b] + 15, jnp.int32(-16))

        ptrs = tuple(jnp.full((16,), start_s[b], jnp.int32) for b in range(NB))
        for c in range(nchunks):
            pltpu.sync_copy(recv.at[pl.ds(ebase + c * cchunk, cchunk)],
                            rbuf.at[pl.ds(0, cchunk)])

            def _scatter_vreg(rv, ev, extra_mask, ptr):
                res = []
                for b in range(NB):
                    m = (rv >= b * rng) & (rv < (b + 1) * rng)
                    if extra_mask is not None:
                        m = m & extra_mask
                    ranks = plsc.cumsum(m.astype(jnp.int32))
                    pos = ptr[b] + ranks - 1
                    packed = lax.shift_left(ev, 13) | (rv - b * rng)
                    plsc.store_scatter(plist, [pos], packed, mask=m)
                    res.append(ptr[b] + jnp.sum(m.astype(jnp.int32)))
                return tuple(res)

            def _part(j, carry):
                rv = rbuf[pl.ds(j * 16, 16)]
                ev = c * cchunk + j * 16 + iot
                return _scatter_vreg(rv, ev, None, carry)
            ptrs = lax.fori_loop(0, nfull, _part, ptrs)
            rv = rbuf[pl.ds(nfull * 16, 16)]
            ev = c * cchunk + nfull * 16 + iot
            ptrs = _scatter_vreg(rv, ev, iot < rem, ptrs)

        for b in range(NB):
            for z in range(rows_pt // zrows):
                pltpu.sync_copy(
                    zbuf, acc.at[pl.ds(sid * rows_pt + z * zrows, zrows)])
            plsc.subcore_barrier()

            cnt_b = cnt_s[b]
            st_b = start_s[b]
            nch = (cnt_b + (K - 1)) // K

            def _chunk(j, carry):
                cb = st_b + j * K
                for g in range(K // 16):
                    pv = plist[pl.ds(cb + 16 * g, 16)]
                    eidc[pl.ds(16 * g, 16)] = (
                        ebase + lax.shift_right_logical(pv, 13))
                    lrecc[pl.ds(16 * g, 16)] = pv & 8191
                pltpu.async_copy(edata.at[eidc], ebuf, sem).wait()
                for g in range(K // 16):
                    rows = 16 * g + iot
                    cols = zi16 + 67
                    sf = plsc.load_gather(ebuf, [rows, cols])
                    sidx[pl.ds(16 * g, 16)] = sf.astype(jnp.int32)
                pltpu.async_copy(ndata.at[sidx], nbuf, sem).wait()

                def _edge(ee, c2):
                    vmsk = jnp.where(j * K + ee < cnt_b, 1.0, 0.0)
                    w0 = ebuf[ee, 0:16] * vmsk
                    w1 = ebuf[ee, 16:32] * vmsk
                    w2 = ebuf[ee, 32:48] * vmsk
                    w3 = ebuf[ee, 48:64] * vmsk
                    yv = ebuf[ee, 56:72]
                    y10 = yv[8]
                    y11 = yv[9]
                    y12 = yv[10]
                    sv = nbuf[ee, 0:16]
                    v0 = nbuf[ee, 16:32]
                    v1 = nbuf[ee, 32:48]
                    v2 = nbuf[ee, 48:64]
                    t = w1 * sv
                    msgbuf[ee, 0:16] = w0 * sv
                    msgbuf[ee, 16:32] = w3 * (v0 * y10 + v1 * y11 + v2 * y12)
                    msgbuf[ee, 32:48] = t * y10
                    msgbuf[ee, 48:64] = w2 * v0
                    msgbuf[ee, 64:80] = t * y11
                    msgbuf[ee, 80:96] = w2 * v1
                    msgbuf[ee, 96:112] = t * y12
                    msgbuf[ee, 112:128] = w2 * v2
                    return c2
                lax.fori_loop(0, K, _edge, 0)
                pltpu.sync_copy(msgbuf, acc.at[lrecc], add=True)
                return carry
            lax.fori_loop(0, nch, _chunk, 0)
            plsc.subcore_barrier()
            for z in range(rows_pt // zrows):
                r0 = sid * rows_pt + z * zrows
                pltpu.sync_copy(acc.at[pl.ds(r0, zrows)],
                                out.at[cid, pl.ds(b * rng + r0, zrows)])
            plsc.subcore_barrier()

    return body(edgedata, receivers, nodedata)


def _prep_body(ns_ref, nvi_ref, out_ref):
    out_ref[:, 0:16] = ns_ref[...]
    out_ref[:, 16:64] = nvi_ref[...]


def _prep_stage(node_scalars, nvi48, block_n):
    n = node_scalars.shape[0]
    grid = n // block_n
    return pl.pallas_call(
        _prep_body,
        grid=(grid,),
        in_specs=[
            pl.BlockSpec((block_n, 16), lambda i: (i, 0)),
            pl.BlockSpec((block_n, 48), lambda i: (i, 0)),
        ],
        out_specs=pl.BlockSpec((block_n, 64), lambda i: (i, 0)),
        out_shape=jax.ShapeDtypeStruct((n, 64), jnp.float32),
    )(node_scalars, nvi48)


def _node_body(pa_ref, pb_ref, ns_ref, nvi_ref, sp_ref,
               w0_ref, w1_ref, sk0_ref, sk1_ref, outs_ref, outv_ref):
    row = pa_ref[...] + pb_ref[...]
    cs = row[:, 0:32]
    sg = cs @ w0_ref[...]
    scal = ns_ref[...]
    spf = sp_ref[...].astype(jnp.float32)
    masks = [jnp.where(spf == float(k), 1.0, 0.0) for k in range(4)]
    for k in range(4):
        sg = sg + masks[k] * (scal @ sk0_ref[k])
    gates = jax.nn.sigmoid(sg[:, MUL:2 * MUL])
    s_out = sg[:, 0:MUL]
    outs_ref[...] = s_out * jax.nn.sigmoid(s_out)
    nvi = nvi_ref[...]
    for i in range(3):
        vi = row[:, 32 + 32 * i:64 + 32 * i] @ w1_ref[...]
        for k in range(4):
            vi = vi + masks[k] * (nvi[:, 16 * i:16 * i + 16] @ sk1_ref[k])
        outv_ref[:, 16 * i:16 * i + 16] = vi * gates


def _node_stage(pa, pb, node_scalars, nvi48, species,
                w0s, w1s, sk0s, sk1s, block_n):
    n = node_scalars.shape[0]
    grid = n // block_n
    return pl.pallas_call(
        _node_body,
        grid=(grid,),
        in_specs=[
            pl.BlockSpec((block_n, 128), lambda i: (i, 0)),
            pl.BlockSpec((block_n, 128), lambda i: (i, 0)),
            pl.BlockSpec((block_n, 16), lambda i: (i, 0)),
            pl.BlockSpec((block_n, 48), lambda i: (i, 0)),
            pl.BlockSpec((block_n, 1), lambda i: (i, 0)),
            pl.BlockSpec((32, 32), lambda i: (0, 0)),
            pl.BlockSpec((32, 16), lambda i: (0, 0)),
            pl.BlockSpec((4, 16, 32), lambda i: (0, 0, 0)),
            pl.BlockSpec((4, 16, 16), lambda i: (0, 0, 0)),
        ],
        out_specs=[
            pl.BlockSpec((block_n, 16), lambda i: (i, 0)),
            pl.BlockSpec((block_n, 48), lambda i: (i, 0)),
        ],
        out_shape=[
            jax.ShapeDtypeStruct((n, 16), jnp.float32),
            jax.ShapeDtypeStruct((n, 48), jnp.float32),
        ],
    )(pa, pb, node_scalars, nvi48, species.reshape(n, 1),
      w0s, w1s, sk0s, sk1s)


def _pick_block(total, want):
    b = min(want, total)
    while total % b:
        b -= 1
    return b


def kernel(node_scalars, node_vectors, vectors, species, senders, receivers,
           mlp_w1, mlp_w2, mlp_w3, lin_w0e, lin_w1o, skip_w0e, skip_w1o):
    e = vectors.shape[0]
    n = node_scalars.shape[0]

    isq3 = 1.0 / np.sqrt(3.0)
    path_scale = np.repeat(np.array([1.0, isq3, 1.0, isq3], np.float32), MUL)
    w3s = mlp_w3 * (EPS * path_scale)[None, :]

    block_e = 3200 if e % 3200 == 0 else e
    edgedata = _edge_stage(vectors, senders, mlp_w1, mlp_w2, w3s, block_e)

    nvi48 = node_vectors.transpose(0, 2, 1).reshape(n, 48)
    block_n = _pick_block(n, 2000)
    nodedata = _prep_stage(node_scalars, nvi48, block_n)

    partials = _sc_conv(edgedata, receivers, nodedata)

    rs2 = np.float32(1.0 / np.sqrt(2.0 * MUL))
    rs1 = np.float32(1.0 / np.sqrt(1.0 * MUL))
    out_s, out_v48 = _node_stage(
        partials[0], partials[1], node_scalars, nvi48, species,
        lin_w0e * rs2, lin_w1o * rs2, skip_w0e * rs1, skip_w1o * rs1,
        block_n)
    out_v = out_v48.reshape(n, 3, MUL).transpose(0, 2, 1)
    return out_s, out_v
